# baseline (speedup 1.0000x reference)
import functools

import jax
import jax.numpy as jnp
from jax import lax
from jax.experimental import pallas as pl
from jax.experimental.pallas import tpu as pltpu
from jax.experimental.pallas import tpu_sc as plsc

_N = 10000
_E = 320000
_DIN = 128
_DOUT = 256

_NC = 2
_NS = 16
_NW = _NC * _NS
_CHUNK = 128
_NCHUNK = 80
_HALF = _NCHUNK // 2
_EPW = _CHUNK * _NCHUNK
_EPAD = _NW * _EPW
_RPAD = 10240
_RPT = _RPAD // _NS
_ZC = 128
_MBLK = 512

_mesh = plsc.VectorSubcoreMesh(core_axis_name="c", subcore_axis_name="s")


@functools.partial(
    pl.kernel,
    mesh=_mesh,
    out_type=jax.ShapeDtypeStruct((_NC, _RPAD, _DIN), jnp.float32),
    scratch_types=[
        pltpu.VMEM((_CHUNK, _DIN), jnp.float32),
        pltpu.VMEM((_CHUNK, _DIN), jnp.float32),
        pltpu.VMEM((_HALF, _CHUNK), jnp.int32),
        pltpu.VMEM((_HALF, _CHUNK), jnp.int32),
        pltpu.VMEM_SHARED((_RPAD, _DIN), jnp.float32),
        pltpu.SemaphoreType.DMA,
        pltpu.SemaphoreType.DMA,
    ],
)
def _sc_segsum(src_hbm, dst_hbm, x_hbm, zeros_hbm, out_hbm,
               rows0_v, rows1_v, src_v, dst_v, acc_sh, sem0, sem1):
    c = lax.axis_index("c")
    s = lax.axis_index("s")
    wid = c * _NS + s
    for k in range(_RPT // _ZC):
        pltpu.sync_copy(zeros_hbm, acc_sh.at[pl.ds(s * _RPT + k * _ZC, _ZC)])
    plsc.subcore_barrier()

    def body(j, carry):
        pltpu.async_copy(x_hbm.at[src_v.at[j]], rows0_v, sem0).wait()
        pltpu.sync_copy(rows0_v, acc_sh.at[dst_v.at[j]], add=True)
        return carry

    for h in range(_NCHUNK // _HALF):
        pltpu.sync_copy(src_hbm.at[wid, pl.ds(h * _HALF, _HALF)], src_v)
        pltpu.sync_copy(dst_hbm.at[wid, pl.ds(h * _HALF, _HALF)], dst_v)
        lax.fori_loop(0, _HALF, body, 0)

    plsc.subcore_barrier()
    for k in range(_RPT // _ZC):
        r0 = s * _RPT + k * _ZC
        pltpu.sync_copy(acc_sh.at[pl.ds(r0, _ZC)], out_hbm.at[c, pl.ds(r0, _ZC)])


def _tc_body(a_ref, w_ref, b_ref, o_ref):
    blk = a_ref[0] + a_ref[1]
    y = jnp.dot(blk, w_ref[...], preferred_element_type=jnp.float32)
    o_ref[...] = jnp.maximum(y + b_ref[...], 0.0)


_tc_matmul = pl.pallas_call(
    _tc_body,
    grid=(_RPAD // _MBLK,),
    in_specs=[
        pl.BlockSpec((_NC, _MBLK, _DIN), lambda i: (0, i, 0)),
        pl.BlockSpec((_DIN, _DOUT), lambda i: (0, 0)),
        pl.BlockSpec((1, _DOUT), lambda i: (0, 0)),
    ],
    out_specs=pl.BlockSpec((_MBLK, _DOUT), lambda i: (i, 0)),
    out_shape=jax.ShapeDtypeStruct((_N, _DOUT), jnp.float32),
)


def kernel(x, edge_index, W, b):
    ei = edge_index.astype(jnp.int32)
    npad = _EPAD - _E
    src = jnp.concatenate([ei[0], jnp.zeros((npad,), jnp.int32)])
    dst = jnp.concatenate([ei[1], jnp.full((npad,), _N, jnp.int32)])
    src = src.reshape(_NW, _NCHUNK, _CHUNK)
    dst = dst.reshape(_NW, _NCHUNK, _CHUNK)
    zeros = jnp.zeros((_ZC, _DIN), jnp.float32)
    acc = _sc_segsum(src, dst, x, zeros)
    return _tc_matmul(acc, W, b.reshape(1, _DOUT))


# speedup vs baseline: 1.0091x; 1.0091x over previous
import functools

import jax
import jax.numpy as jnp
from jax import lax
from jax.experimental import pallas as pl
from jax.experimental.pallas import tpu as pltpu
from jax.experimental.pallas import tpu_sc as plsc

_N = 10000
_E = 320000
_DIN = 128
_DOUT = 256

_NC = 2
_NS = 16
_NW = _NC * _NS
_CHUNK = 80
_NCHUNK = 128
_HALF = _NCHUNK // 2
_EPW = _CHUNK * _NCHUNK
_EPAD = _NW * _EPW
_RPAD = 10240
_RPT = _RPAD // _NS
_ZC = 128
_MBLK = 512

_mesh = plsc.VectorSubcoreMesh(core_axis_name="c", subcore_axis_name="s")


@functools.partial(
    pl.kernel,
    mesh=_mesh,
    out_type=jax.ShapeDtypeStruct((_NC, _RPAD, _DIN), jnp.float32),
    scratch_types=[
        pltpu.VMEM((_CHUNK, _DIN), jnp.float32),
        pltpu.VMEM((_CHUNK, _DIN), jnp.float32),
        pltpu.VMEM((_HALF, _CHUNK), jnp.int32),
        pltpu.VMEM((_HALF, _CHUNK), jnp.int32),
        pltpu.VMEM_SHARED((_RPAD, _DIN), jnp.float32),
        pltpu.SemaphoreType.DMA,
        pltpu.SemaphoreType.DMA,
    ],
)
def _sc_segsum(src_hbm, dst_hbm, x_hbm, zeros_hbm, out_hbm,
               rows0_v, rows1_v, src_v, dst_v, acc_sh, sem0, sem1):
    c = lax.axis_index("c")
    s = lax.axis_index("s")
    wid = c * _NS + s
    for k in range(_RPT // _ZC):
        pltpu.sync_copy(zeros_hbm, acc_sh.at[pl.ds(s * _RPT + k * _ZC, _ZC)])
    plsc.subcore_barrier()

    def body(g, carry):
        j0 = 2 * g
        j1 = 2 * g + 1
        h0 = pltpu.async_copy(x_hbm.at[src_v.at[j0]], rows0_v, sem0)
        h1 = pltpu.async_copy(x_hbm.at[src_v.at[j1]], rows1_v, sem1)
        h0.wait()
        pltpu.sync_copy(rows0_v, acc_sh.at[dst_v.at[j0]], add=True)
        h1.wait()
        pltpu.sync_copy(rows1_v, acc_sh.at[dst_v.at[j1]], add=True)
        return carry

    for h in range(_NCHUNK // _HALF):
        pltpu.sync_copy(src_hbm.at[wid, pl.ds(h * _HALF, _HALF)], src_v)
        pltpu.sync_copy(dst_hbm.at[wid, pl.ds(h * _HALF, _HALF)], dst_v)
        lax.fori_loop(0, _HALF // 2, body, 0)

    plsc.subcore_barrier()
    for k in range(_RPT // _ZC):
        r0 = s * _RPT + k * _ZC
        pltpu.sync_copy(acc_sh.at[pl.ds(r0, _ZC)], out_hbm.at[c, pl.ds(r0, _ZC)])


def _tc_body(a_ref, w_ref, b_ref, o_ref):
    blk = a_ref[0] + a_ref[1]
    y = jnp.dot(blk, w_ref[...], preferred_element_type=jnp.float32)
    o_ref[...] = jnp.maximum(y + b_ref[...], 0.0)


_tc_matmul = pl.pallas_call(
    _tc_body,
    grid=(_RPAD // _MBLK,),
    in_specs=[
        pl.BlockSpec((_NC, _MBLK, _DIN), lambda i: (0, i, 0)),
        pl.BlockSpec((_DIN, _DOUT), lambda i: (0, 0)),
        pl.BlockSpec((1, _DOUT), lambda i: (0, 0)),
    ],
    out_specs=pl.BlockSpec((_MBLK, _DOUT), lambda i: (i, 0)),
    out_shape=jax.ShapeDtypeStruct((_N, _DOUT), jnp.float32),
)


def kernel(x, edge_index, W, b):
    ei = edge_index.astype(jnp.int32)
    npad = _EPAD - _E
    src = jnp.concatenate([ei[0], jnp.zeros((npad,), jnp.int32)])
    dst = jnp.concatenate([ei[1], jnp.full((npad,), _N, jnp.int32)])
    src = src.reshape(_NW, _NCHUNK, _CHUNK)
    dst = dst.reshape(_NW, _NCHUNK, _CHUNK)
    zeros = jnp.zeros((_ZC, _DIN), jnp.float32)
    acc = _sc_segsum(src, dst, x, zeros)
    return _tc_matmul(acc, W, b.reshape(1, _DOUT))


# speedup vs baseline: 1.0093x; 1.0002x over previous
import functools

import jax
import jax.numpy as jnp
from jax import lax
from jax.experimental import pallas as pl
from jax.experimental.pallas import tpu as pltpu
from jax.experimental.pallas import tpu_sc as plsc

_N = 10000
_E = 320000
_DIN = 128
_DOUT = 256

_NC = 2
_NS = 16
_NW = _NC * _NS
_CHUNK = 80
_NCHUNK = 128
_HALF = _NCHUNK // 2
_EPW = _CHUNK * _NCHUNK
_EPAD = _NW * _EPW
_RPAD = 10240
_RPT = _RPAD // _NS
_ZC = 128
_MBLK = 512

_mesh = plsc.VectorSubcoreMesh(core_axis_name="c", subcore_axis_name="s")


@functools.partial(
    pl.kernel,
    mesh=_mesh,
    out_type=jax.ShapeDtypeStruct((_NC, _RPAD, _DIN), jnp.float32),
    scratch_types=[
        pltpu.VMEM((_CHUNK, _DIN), jnp.float32),
        pltpu.VMEM((_CHUNK, _DIN), jnp.float32),
        pltpu.VMEM((_HALF, _CHUNK), jnp.int32),
        pltpu.VMEM((_HALF, _CHUNK), jnp.int32),
        pltpu.VMEM_SHARED((_RPAD, _DIN), jnp.float32),
        pltpu.SemaphoreType.DMA,
        pltpu.SemaphoreType.DMA,
    ],
)
def _sc_segsum(src_hbm, dst_hbm, x_hbm, zeros_hbm, out_hbm,
               rows0_v, rows1_v, src_v, dst_v, acc_sh, sem0, sem1):
    c = lax.axis_index("c")
    s = lax.axis_index("s")
    wid = c * _NS + s
    for k in range(_RPT // _ZC):
        pltpu.sync_copy(zeros_hbm, acc_sh.at[pl.ds(s * _RPT + k * _ZC, _ZC)])
    plsc.subcore_barrier()

    def body(g, carry):
        j0 = 2 * g
        j1 = 2 * g + 1
        h0 = pltpu.async_copy(x_hbm.at[src_v.at[j0]], rows0_v, sem0)
        h1 = pltpu.async_copy(x_hbm.at[src_v.at[j1]], rows1_v, sem1)
        h0.wait()
        pltpu.sync_copy(rows0_v, acc_sh.at[dst_v.at[j0]], add=True)
        h1.wait()
        pltpu.sync_copy(rows1_v, acc_sh.at[dst_v.at[j1]], add=True)
        return carry

    for h in range(_NCHUNK // _HALF):
        pltpu.sync_copy(src_hbm.at[wid, pl.ds(h * _HALF, _HALF)], src_v)
        pltpu.sync_copy(dst_hbm.at[wid, pl.ds(h * _HALF, _HALF)], dst_v)
        lax.fori_loop(0, _HALF // 2, body, 0)

    plsc.subcore_barrier()
    for k in range(_RPT // _ZC):
        r0 = s * _RPT + k * _ZC
        pltpu.sync_copy(acc_sh.at[pl.ds(r0, _ZC)], out_hbm.at[c, pl.ds(r0, _ZC)])


def _tc_body(a_ref, w_ref, b_ref, o_ref):
    blk = a_ref[0] + a_ref[1]
    y = jnp.dot(blk, w_ref[...], preferred_element_type=jnp.float32)
    o_ref[...] = jnp.maximum(y + b_ref[...], 0.0)


_tc_matmul = pl.pallas_call(
    _tc_body,
    grid=(_RPAD // _MBLK,),
    in_specs=[
        pl.BlockSpec((_NC, _MBLK, _DIN), lambda i: (0, i, 0)),
        pl.BlockSpec((_DIN, _DOUT), lambda i: (0, 0)),
        pl.BlockSpec((1, _DOUT), lambda i: (0, 0)),
    ],
    out_specs=pl.BlockSpec((_MBLK, _DOUT), lambda i: (i, 0)),
    out_shape=jax.ShapeDtypeStruct((_N, _DOUT), jnp.float32),
)


def kernel(x, edge_index, W, b):
    ei = edge_index.astype(jnp.int32)
    npad = _EPAD - _E
    src = jnp.concatenate([ei[0], jnp.zeros((npad,), jnp.int32)])
    trash = _N + (jnp.arange(npad, dtype=jnp.int32) % (_RPAD - _N))
    dst = jnp.concatenate([ei[1], trash])
    src = src.reshape(_NW, _NCHUNK, _CHUNK)
    dst = dst.reshape(_NW, _NCHUNK, _CHUNK)
    zeros = jnp.zeros((_ZC, _DIN), jnp.float32)
    acc = _sc_segsum(src, dst, x, zeros)
    return _tc_matmul(acc, W, b.reshape(1, _DOUT))


# speedup vs baseline: 3.0165x; 2.9886x over previous
import functools

import jax
import jax.numpy as jnp
from jax import lax
from jax.experimental import pallas as pl
from jax.experimental.pallas import tpu as pltpu
from jax.experimental.pallas import tpu_sc as plsc

_N = 10000
_E = 320000
_DIN = 128
_DOUT = 256

_NC = 2
_NS = 16
_NW = _NC * _NS
_CHUNK = 80
_NCHUNK = 125
_EPW = _CHUNK * _NCHUNK
_RPAD = 10240
_RPT = _RPAD // _NS
_ZC = 128
_MBLK = 512

_mesh = plsc.VectorSubcoreMesh(core_axis_name="c", subcore_axis_name="s")


@functools.partial(
    pl.kernel,
    mesh=_mesh,
    out_type=jax.ShapeDtypeStruct((_NC, _RPAD, _DIN), jnp.float32),
    scratch_types=[
        pltpu.VMEM((_CHUNK, _DIN), jnp.float32),
        pltpu.VMEM((_CHUNK, _DIN), jnp.float32),
        pltpu.VMEM((_EPW,), jnp.int32),
        pltpu.VMEM((_NCHUNK, _CHUNK), jnp.int32),
        pltpu.VMEM_SHARED((_RPAD, _DIN), jnp.float32),
        pltpu.SemaphoreType.DMA,
        pltpu.SemaphoreType.DMA,
    ],
)
def _sc_segsum(src_hbm, dst_hbm, x_hbm, zeros_hbm, out_hbm,
               rows0_v, rows1_v, src_v, dst_v, acc_sh, sem0, sem1):
    c = lax.axis_index("c")
    s = lax.axis_index("s")
    wid = c * _NS + s
    pltpu.sync_copy(src_hbm.at[wid], src_v)
    pltpu.sync_copy(dst_hbm.at[wid], dst_v)
    for k in range(_RPT // _ZC):
        pltpu.sync_copy(zeros_hbm, acc_sh.at[pl.ds(s * _RPT + k * _ZC, _ZC)])
    plsc.subcore_barrier()

    def body(g, carry):
        j0 = 2 * g
        j1 = 2 * g + 1
        h0 = pltpu.async_copy(x_hbm.at[src_v.at[pl.ds(j0 * _CHUNK, _CHUNK)]],
                              rows0_v, sem0)
        h1 = pltpu.async_copy(x_hbm.at[src_v.at[pl.ds(j1 * _CHUNK, _CHUNK)]],
                              rows1_v, sem1)
        h0.wait()
        pltpu.sync_copy(rows0_v, acc_sh.at[dst_v.at[j0]], add=True)
        h1.wait()
        pltpu.sync_copy(rows1_v, acc_sh.at[dst_v.at[j1]], add=True)
        return carry

    lax.fori_loop(0, _NCHUNK // 2, body, 0)
    jl = _NCHUNK - 1
    pltpu.async_copy(x_hbm.at[src_v.at[pl.ds(jl * _CHUNK, _CHUNK)]],
                     rows0_v, sem0).wait()
    pltpu.sync_copy(rows0_v, acc_sh.at[dst_v.at[jl]], add=True)

    plsc.subcore_barrier()
    for k in range(_RPT // _ZC):
        r0 = s * _RPT + k * _ZC
        pltpu.sync_copy(acc_sh.at[pl.ds(r0, _ZC)], out_hbm.at[c, pl.ds(r0, _ZC)])


def _tc_body(a_ref, w_ref, b_ref, o_ref):
    blk = a_ref[0] + a_ref[1]
    y = jnp.dot(blk, w_ref[...], preferred_element_type=jnp.float32)
    o_ref[...] = jnp.maximum(y + b_ref[...], 0.0)


_tc_matmul = pl.pallas_call(
    _tc_body,
    grid=(_RPAD // _MBLK,),
    in_specs=[
        pl.BlockSpec((_NC, _MBLK, _DIN), lambda i: (0, i, 0)),
        pl.BlockSpec((_DIN, _DOUT), lambda i: (0, 0)),
        pl.BlockSpec((1, _DOUT), lambda i: (0, 0)),
    ],
    out_specs=pl.BlockSpec((_MBLK, _DOUT), lambda i: (i, 0)),
    out_shape=jax.ShapeDtypeStruct((_N, _DOUT), jnp.float32),
)


def kernel(x, edge_index, W, b):
    ei = edge_index.astype(jnp.int32)
    src = ei[0].reshape(_NW, _EPW)
    dst = ei[1].reshape(_NW, _NCHUNK, _CHUNK)
    zeros = jnp.zeros((_ZC, _DIN), jnp.float32)
    acc = _sc_segsum(src, dst, x, zeros)
    return _tc_matmul(acc, W, b.reshape(1, _DOUT))


# speedup vs baseline: 3.7408x; 1.2401x over previous
import functools

import jax
import jax.numpy as jnp
from jax import lax
from jax.experimental import pallas as pl
from jax.experimental.pallas import tpu as pltpu
from jax.experimental.pallas import tpu_sc as plsc

_N = 10000
_E = 320000
_DIN = 128
_DOUT = 256

_NC = 2
_NS = 16
_NW = _NC * _NS
_CHUNK = 80
_NCHUNK = 125
_EPW = _CHUNK * _NCHUNK
_RPAD = 10240
_RPT = _RPAD // _NS
_ZC = 128
_MBLK = 512

_mesh = plsc.VectorSubcoreMesh(core_axis_name="c", subcore_axis_name="s")


@functools.partial(
    pl.kernel,
    mesh=_mesh,
    out_type=jax.ShapeDtypeStruct((_NC, _RPAD, _DIN), jnp.float32),
    scratch_types=[
        pltpu.VMEM((_CHUNK, _DIN), jnp.float32),
        pltpu.VMEM((_CHUNK, _DIN), jnp.float32),
        pltpu.VMEM((_EPW,), jnp.int32),
        pltpu.VMEM((_NCHUNK, _CHUNK), jnp.int32),
        pltpu.VMEM_SHARED((_RPAD, _DIN), jnp.float32),
        pltpu.SemaphoreType.DMA,
        pltpu.SemaphoreType.DMA,
    ],
)
def _sc_segsum(src_hbm, dst_hbm, x_hbm, zeros_hbm, out_hbm,
               rows0_v, rows1_v, src_v, dst_v, acc_sh, sem0, sem1):
    c = lax.axis_index("c")
    s = lax.axis_index("s")
    wid = c * _NS + s
    pltpu.sync_copy(src_hbm.at[wid], src_v)
    pltpu.sync_copy(dst_hbm.at[wid], dst_v)
    for k in range(_RPT // _ZC):
        pltpu.sync_copy(zeros_hbm, acc_sh.at[pl.ds(s * _RPT + k * _ZC, _ZC)])
    plsc.subcore_barrier()

    def _gather(j, buf, sem):
        pltpu.async_copy(x_hbm.at[src_v.at[pl.ds(j * _CHUNK, _CHUNK)]],
                         buf, sem)

    def _gwait(j, buf, sem):
        pltpu.make_async_copy(x_hbm.at[src_v.at[pl.ds(j * _CHUNK, _CHUNK)]],
                              buf, sem).wait()

    def _scat(j, buf):
        pltpu.sync_copy(buf, acc_sh.at[dst_v.at[j]], add=True)

    _gather(0, rows0_v, sem0)
    _gather(1, rows1_v, sem1)

    def body(g, carry):
        j0 = 2 * g
        _gwait(j0, rows0_v, sem0)
        _scat(j0, rows0_v)
        _gather(j0 + 2, rows0_v, sem0)
        _gwait(j0 + 1, rows1_v, sem1)
        _scat(j0 + 1, rows1_v)
        _gather(j0 + 3, rows1_v, sem1)
        return carry

    lax.fori_loop(0, 60, body, 0)
    _gwait(120, rows0_v, sem0)
    _scat(120, rows0_v)
    _gather(122, rows0_v, sem0)
    _gwait(121, rows1_v, sem1)
    _scat(121, rows1_v)
    _gather(123, rows1_v, sem1)
    _gwait(122, rows0_v, sem0)
    _scat(122, rows0_v)
    _gather(124, rows0_v, sem0)
    _gwait(123, rows1_v, sem1)
    _scat(123, rows1_v)
    _gwait(124, rows0_v, sem0)
    _scat(124, rows0_v)

    plsc.subcore_barrier()
    for k in range(_RPT // _ZC):
        r0 = s * _RPT + k * _ZC
        pltpu.sync_copy(acc_sh.at[pl.ds(r0, _ZC)], out_hbm.at[c, pl.ds(r0, _ZC)])


def _tc_body(a_ref, w_ref, b_ref, o_ref):
    blk = a_ref[0] + a_ref[1]
    y = jnp.dot(blk, w_ref[...], preferred_element_type=jnp.float32)
    o_ref[...] = jnp.maximum(y + b_ref[...], 0.0)


_tc_matmul = pl.pallas_call(
    _tc_body,
    grid=(_RPAD // _MBLK,),
    in_specs=[
        pl.BlockSpec((_NC, _MBLK, _DIN), lambda i: (0, i, 0)),
        pl.BlockSpec((_DIN, _DOUT), lambda i: (0, 0)),
        pl.BlockSpec((1, _DOUT), lambda i: (0, 0)),
    ],
    out_specs=pl.BlockSpec((_MBLK, _DOUT), lambda i: (i, 0)),
    out_shape=jax.ShapeDtypeStruct((_N, _DOUT), jnp.float32),
)


def kernel(x, edge_index, W, b):
    ei = edge_index.astype(jnp.int32)
    src = ei[0].reshape(_NW, _EPW)
    dst = ei[1].reshape(_NW, _NCHUNK, _CHUNK)
    zeros = jnp.zeros((_ZC, _DIN), jnp.float32)
    acc = _sc_segsum(src, dst, x, zeros)
    return _tc_matmul(acc, W, b.reshape(1, _DOUT))


# speedup vs baseline: 3.8480x; 1.0287x over previous
import functools

import jax
import jax.numpy as jnp
from jax import lax
from jax.experimental import pallas as pl
from jax.experimental.pallas import tpu as pltpu
from jax.experimental.pallas import tpu_sc as plsc

_N = 10000
_E = 320000
_DIN = 128
_DOUT = 256

_NC = 2
_NS = 16
_NW = _NC * _NS
_CHUNK = 80
_NCHUNK = 125
_EPW = _CHUNK * _NCHUNK
_RPAD = 10240
_RPT = _RPAD // _NS
_ZC = 128
_MBLK = 1024

_mesh = plsc.VectorSubcoreMesh(core_axis_name="c", subcore_axis_name="s")


@functools.partial(
    pl.kernel,
    mesh=_mesh,
    out_type=jax.ShapeDtypeStruct((_NC, _RPAD, _DIN), jnp.float32),
    scratch_types=[
        pltpu.VMEM((_CHUNK, _DIN), jnp.float32),
        pltpu.VMEM((_CHUNK, _DIN), jnp.float32),
        pltpu.VMEM((_EPW,), jnp.int32),
        pltpu.VMEM((_NCHUNK, _CHUNK), jnp.int32),
        pltpu.VMEM_SHARED((_RPAD, _DIN), jnp.float32),
        pltpu.SemaphoreType.DMA,
        pltpu.SemaphoreType.DMA,
    ],
)
def _sc_segsum(src_hbm, dst_hbm, x_hbm, zeros_hbm, out_hbm,
               rows0_v, rows1_v, src_v, dst_v, acc_sh, sem0, sem1):
    c = lax.axis_index("c")
    s = lax.axis_index("s")
    wid = c * _NS + s
    pltpu.sync_copy(src_hbm.at[wid], src_v)
    pltpu.sync_copy(dst_hbm.at[wid], dst_v)
    for k in range(_RPT // _ZC):
        pltpu.sync_copy(zeros_hbm, acc_sh.at[pl.ds(s * _RPT + k * _ZC, _ZC)])
    plsc.subcore_barrier()

    def _gather(j, buf, sem):
        pltpu.async_copy(x_hbm.at[src_v.at[pl.ds(j * _CHUNK, _CHUNK)]],
                         buf, sem)

    def _gwait(j, buf, sem):
        pltpu.make_async_copy(x_hbm.at[src_v.at[pl.ds(j * _CHUNK, _CHUNK)]],
                              buf, sem).wait()

    def _scat(j, buf):
        pltpu.sync_copy(buf, acc_sh.at[dst_v.at[j]], add=True)

    _gather(0, rows0_v, sem0)
    _gather(1, rows1_v, sem1)

    def body(g, carry):
        j0 = 2 * g
        _gwait(j0, rows0_v, sem0)
        _scat(j0, rows0_v)
        _gather(j0 + 2, rows0_v, sem0)
        _gwait(j0 + 1, rows1_v, sem1)
        _scat(j0 + 1, rows1_v)
        _gather(j0 + 3, rows1_v, sem1)
        return carry

    lax.fori_loop(0, 60, body, 0)
    _gwait(120, rows0_v, sem0)
    _scat(120, rows0_v)
    _gather(122, rows0_v, sem0)
    _gwait(121, rows1_v, sem1)
    _scat(121, rows1_v)
    _gather(123, rows1_v, sem1)
    _gwait(122, rows0_v, sem0)
    _scat(122, rows0_v)
    _gather(124, rows0_v, sem0)
    _gwait(123, rows1_v, sem1)
    _scat(123, rows1_v)
    _gwait(124, rows0_v, sem0)
    _scat(124, rows0_v)

    plsc.subcore_barrier()
    for k in range(_RPT // _ZC):
        r0 = s * _RPT + k * _ZC
        pltpu.sync_copy(acc_sh.at[pl.ds(r0, _ZC)], out_hbm.at[c, pl.ds(r0, _ZC)])


def _tc_body(a_ref, w_ref, b_ref, o_ref):
    blk = a_ref[0] + a_ref[1]
    y = jnp.dot(blk, w_ref[...], preferred_element_type=jnp.float32)
    o_ref[...] = jnp.maximum(y + b_ref[...], 0.0)


_tc_matmul = pl.pallas_call(
    _tc_body,
    grid=(_RPAD // _MBLK,),
    in_specs=[
        pl.BlockSpec((_NC, _MBLK, _DIN), lambda i: (0, i, 0)),
        pl.BlockSpec((_DIN, _DOUT), lambda i: (0, 0)),
        pl.BlockSpec((1, _DOUT), lambda i: (0, 0)),
    ],
    out_specs=pl.BlockSpec((_MBLK, _DOUT), lambda i: (i, 0)),
    out_shape=jax.ShapeDtypeStruct((_N, _DOUT), jnp.float32),
)


def kernel(x, edge_index, W, b):
    ei = edge_index.astype(jnp.int32)
    src = ei[0].reshape(_NW, _EPW)
    dst = ei[1].reshape(_NW, _NCHUNK, _CHUNK)
    zeros = jnp.zeros((_ZC, _DIN), jnp.float32)
    acc = _sc_segsum(src, dst, x, zeros)
    return _tc_matmul(acc, W, b.reshape(1, _DOUT))
